# baseline (speedup 1.0000x reference)
import functools

import jax
import jax.numpy as jnp
from jax import lax
from jax.experimental import pallas as pl
from jax.experimental.pallas import tpu as pltpu
from jax.experimental.pallas import tpu_sc as plsc

_N, _D = 128, 32768
_NC, _NS = 2, 16
_NW = _NC * _NS
_RPW = _N // _NW
_K = 8
_CHW = _D // _K
_NDMA = 4


def _sc_argmax_body(x_hbm, out_hbm, buf, accv, sem0, sem1):
    c = lax.axis_index("c")
    s = lax.axis_index("s")
    w = s * _NC + c
    row0 = w * _RPW
    sems = (sem0, sem1)
    lane = lax.iota(jnp.int32, 16)
    acc = jnp.zeros((16,), jnp.int32)

    def start_row(r, b):
        seg = _D // _NDMA
        return [
            pltpu.make_async_copy(
                x_hbm.at[row0 + r, pl.ds(j * seg, seg)],
                buf.at[b, pl.ds(j * seg, seg)],
                sems[b],
            )
            for j in range(_NDMA)
        ]

    def launch(copies):
        for cc in copies:
            cc.start()

    cp = start_row(0, 0)
    launch(cp)
    for r in range(_RPW):
        b = r % 2
        if r + 1 < _RPW:
            nxt = start_row(r + 1, (r + 1) % 2)
            launch(nxt)
        for cc in cp:
            cc.wait()

        def step(i, carry):
            vmaxs, vidxs = carry
            ib = jnp.full((16,), i, jnp.int32)
            nmaxs, nidxs = [], []
            for k in range(_K):
                v = buf[b, pl.ds(k * _CHW + i * 16, 16)]
                upd = v > vmaxs[k]
                nmaxs.append(jnp.where(upd, v, vmaxs[k]))
                nidxs.append(jnp.where(upd, ib, vidxs[k]))
            return tuple(nmaxs), tuple(nidxs)

        vmaxs, vidxs = lax.fori_loop(
            0,
            _D // 16 // _K,
            step,
            (
                tuple(jnp.full((16,), -jnp.inf, jnp.float32) for _ in range(_K)),
                tuple(jnp.zeros((16,), jnp.int32) for _ in range(_K)),
            ),
            unroll=4,
        )
        m16 = functools.reduce(jnp.maximum, vmaxs)
        m = jnp.max(m16)
        big = jnp.iinfo(jnp.int32).max
        cand = functools.reduce(
            jnp.minimum,
            [
                jnp.where(
                    vmaxs[k] == m,
                    (vidxs[k] + k * (_CHW // 16)) * 16 + lane,
                    big,
                )
                for k in range(_K)
            ],
        )
        g = jnp.min(cand)
        acc = jnp.where(lane == r, g, acc)
        if r + 1 < _RPW:
            cp = nxt
    accv[...] = acc
    pltpu.sync_copy(accv, out_hbm.at[w])


_sc_argmax = pl.kernel(
    _sc_argmax_body,
    out_type=jax.ShapeDtypeStruct((_NW, 16), jnp.int32),
    mesh=plsc.VectorSubcoreMesh(core_axis_name="c", subcore_axis_name="s"),
    compiler_params=pltpu.CompilerParams(needs_layout_passes=False),
    scratch_types=[
        pltpu.VMEM((2, _D), jnp.float32),
        pltpu.VMEM((16,), jnp.int32),
        pltpu.SemaphoreType.DMA,
        pltpu.SemaphoreType.DMA,
    ],
)


def kernel(logits):
    idx = _sc_argmax(logits)[:, :_RPW].reshape(_N)
    col = jnp.arange(_D, dtype=jnp.int32)[None, :]
    inf = jnp.float32(jnp.inf)
    return jnp.where(col == idx[:, None], inf, -inf)


# speedup vs baseline: 3.6633x; 3.6633x over previous
import jax
import jax.numpy as jnp
from jax.experimental import pallas as pl

_ROWS = 64


def _hardmax_block(x_ref, o_ref):
    x = x_ref[...]
    idx = jnp.argmax(x, axis=1)[:, None]
    col = jax.lax.broadcasted_iota(jnp.int32, x.shape, 1)
    inf = jnp.float32(jnp.inf)
    o_ref[...] = jnp.where(col == idx, inf, -inf)


def kernel(logits):
    n, d = logits.shape
    return pl.pallas_call(
        _hardmax_block,
        grid=(n // _ROWS,),
        in_specs=[pl.BlockSpec((_ROWS, d), lambda i: (i, 0))],
        out_specs=pl.BlockSpec((_ROWS, d), lambda i: (i, 0)),
        out_shape=jax.ShapeDtypeStruct((n, d), jnp.float32),
    )(logits)


# speedup vs baseline: 3.7323x; 1.0188x over previous
import jax
import jax.numpy as jnp
from jax.experimental import pallas as pl

_ROWS = 64


def _hardmax_block(x_ref, o_ref):
    x = x_ref[...]
    m = jnp.max(x, axis=1, keepdims=True)
    col = jax.lax.broadcasted_iota(jnp.int32, x.shape, 1)
    cand = jnp.where(x == m, col, jnp.iinfo(jnp.int32).max)
    idx = jnp.min(cand, axis=1, keepdims=True)
    inf = jnp.float32(jnp.inf)
    o_ref[...] = jnp.where(col == idx, inf, -inf)


def kernel(logits):
    n, d = logits.shape
    return pl.pallas_call(
        _hardmax_block,
        grid=(n // _ROWS,),
        in_specs=[pl.BlockSpec((_ROWS, d), lambda i: (i, 0))],
        out_specs=pl.BlockSpec((_ROWS, d), lambda i: (i, 0)),
        out_shape=jax.ShapeDtypeStruct((n, d), jnp.float32),
    )(logits)
